# Initial kernel scaffold; baseline (speedup 1.0000x reference)
import jax
import jax.numpy as jnp
from jax.experimental import pallas as pl


def kernel(x, edge_index, W0, b0, W1, b1, Wg, bg):
    raise NotImplementedError("write your pallas kernel here")


# speedup vs baseline: 5.4430x; 5.4430x over previous
import functools

import jax
import jax.numpy as jnp
from jax import lax
from jax.experimental import pallas as pl
from jax.experimental.pallas import tpu as pltpu
from jax.experimental.pallas import tpu_sc as plsc

N = 10000
E = 160000
D_IN = 256
H = 512
D_OUT = 256

NPAD = 10240
RPT = 640
B = 80
NB = 125
RB = 2000
CW = 128

_MESH = dict(core_axis_name="c", subcore_axis_name="s")


def _sc_degrees(ei2):
    @functools.partial(
        pl.kernel,
        out_type=jax.ShapeDtypeStruct((2 * NPAD,), jnp.float32),
        mesh=plsc.VectorSubcoreMesh(**_MESH),
        scratch_types=[
            pltpu.VMEM((NB, B), jnp.int32),
            pltpu.VMEM((B,), jnp.float32),
            pltpu.VMEM((RPT,), jnp.float32),
            pltpu.VMEM_SHARED((NPAD,), jnp.float32),
        ],
    )
    def deg_kernel(ei_hbm, deg_hbm, idx, ones, fbuf, acc):
        cid = lax.axis_index("c")
        sid = lax.axis_index("s")

        pltpu.sync_copy(ei_hbm.at[cid, sid], idx)

        def fill_ones(j, _):
            ones[pl.ds(j * 16, 16)] = jnp.full((16,), 1.0, jnp.float32)
            return 0
        lax.fori_loop(0, B // 16, fill_ones, 0)

        def fill_zero(j, _):
            fbuf[pl.ds(j * 16, 16)] = jnp.zeros((16,), jnp.float32)
            return 0
        lax.fori_loop(0, RPT // 16, fill_zero, 0)

        r0 = sid * RPT
        pltpu.sync_copy(fbuf, acc.at[pl.ds(r0, RPT)])
        plsc.subcore_barrier()

        def batch(j, _):
            pltpu.sync_copy(ones, acc.at[idx.at[j]], add=True)
            return 0
        lax.fori_loop(0, NB, batch, 0)
        plsc.subcore_barrier()

        pltpu.sync_copy(acc.at[pl.ds(r0, RPT)], fbuf)
        pltpu.sync_copy(fbuf, deg_hbm.at[pl.ds(cid * NPAD + r0, RPT)])

    return deg_kernel(ei2)


def _sc_aggregate(ei2, tab, nc):
    ncps = nc // 2

    @functools.partial(
        pl.kernel,
        out_type=jax.ShapeDtypeStruct((nc * N, CW), jnp.float32),
        mesh=plsc.VectorSubcoreMesh(**_MESH),
        scratch_types=[
            pltpu.VMEM((NB, B), jnp.int32),
            pltpu.VMEM((NB, B), jnp.int32),
            pltpu.VMEM((B, CW), jnp.float32),
            pltpu.VMEM_SHARED((NPAD, CW), jnp.float32),
            pltpu.SemaphoreType.DMA,
        ],
    )
    def agg_kernel(ei_hbm, tab_hbm, out_hbm, idx_s, idx_d, rows, acc, sem):
        cid = lax.axis_index("c")
        sid = lax.axis_index("s")

        pltpu.sync_copy(ei_hbm.at[0, sid], idx_s)
        pltpu.sync_copy(ei_hbm.at[1, sid], idx_d)

        def add_off(off):
            def body(r, _):
                for k in range(B // 16):
                    sl = (r, pl.ds(k * 16, 16))
                    idx_s[sl] = idx_s[sl] + off
                return 0
            lax.fori_loop(0, NB, body, 0)

        add_off(cid * (ncps * N))

        def fill_zero(j, _):
            for k in range(CW // 16):
                rows[j, pl.ds(k * 16, 16)] = jnp.zeros((16,), jnp.float32)
            return 0

        r0 = sid * RPT
        for ci in range(ncps):
            lax.fori_loop(0, B, fill_zero, 0)
            for blk in range(RPT // B):
                pltpu.sync_copy(rows, acc.at[pl.ds(r0 + blk * B, B)])
            plsc.subcore_barrier()

            def batch(j, _):
                pltpu.async_copy(tab_hbm.at[idx_s.at[j]], rows, sem).wait()
                pltpu.sync_copy(rows, acc.at[idx_d.at[j]], add=True)
                return 0
            lax.fori_loop(0, NB, batch, 0)
            plsc.subcore_barrier()

            chunk_base = (cid * ncps + ci) * N
            for blk in range(RPT // B):
                rr = r0 + blk * B

                @pl.when(rr < N)
                def _():
                    pltpu.sync_copy(acc.at[pl.ds(rr, B)], rows)
                    pltpu.sync_copy(rows, out_hbm.at[pl.ds(chunk_base + rr, B)])

            if ci + 1 < ncps:
                add_off(N)

    return agg_kernel(ei2, tab)


def _tc_scale(x, dout_col):
    nc = D_IN // CW

    def body(x_ref, d_ref, o_ref):
        nrm = lax.rsqrt(jnp.maximum(d_ref[...], 1.0))
        xs = x_ref[...] * nrm
        for c in range(nc):
            o_ref[c] = xs[:, c * CW:(c + 1) * CW]

    return pl.pallas_call(
        body,
        grid=(N // RB,),
        in_specs=[
            pl.BlockSpec((RB, D_IN), lambda i: (i, 0)),
            pl.BlockSpec((RB, 1), lambda i: (i, 0)),
        ],
        out_specs=pl.BlockSpec((nc, RB, CW), lambda i: (0, i, 0)),
        out_shape=jax.ShapeDtypeStruct((nc, N, CW), jnp.float32),
    )(x, dout_col)


def _tc_layer1(agg, din_col, dout_col, W0, b0r):
    nci = D_IN // CW
    nco = H // CW

    def body(a_ref, din_ref, dout_ref, w_ref, b_ref, o_ref):
        nd = lax.rsqrt(jnp.maximum(din_ref[...], 1.0))
        ns = lax.rsqrt(jnp.maximum(dout_ref[...], 1.0))
        h = jnp.dot(a_ref[0] * nd, w_ref[0:CW, :],
                    preferred_element_type=jnp.float32)
        for c in range(1, nci):
            h = h + jnp.dot(a_ref[c] * nd, w_ref[c * CW:(c + 1) * CW, :],
                            preferred_element_type=jnp.float32)
        h = jnp.maximum(h + b_ref[...], 0.0) * ns
        for c in range(nco):
            o_ref[c] = h[:, c * CW:(c + 1) * CW]

    return pl.pallas_call(
        body,
        grid=(N // RB,),
        in_specs=[
            pl.BlockSpec((nci, RB, CW), lambda i: (0, i, 0)),
            pl.BlockSpec((RB, 1), lambda i: (i, 0)),
            pl.BlockSpec((RB, 1), lambda i: (i, 0)),
            pl.BlockSpec((D_IN, H), lambda i: (0, 0)),
            pl.BlockSpec((1, H), lambda i: (0, 0)),
        ],
        out_specs=pl.BlockSpec((nco, RB, CW), lambda i: (0, i, 0)),
        out_shape=jax.ShapeDtypeStruct((nco, N, CW), jnp.float32),
    )(agg, din_col, dout_col, W0, b0r)


def _tc_readout(agg, din_col, W1, b1r, Wg, bgr):
    nblk = N // RB

    nci = H // CW

    def body(a_ref, din_ref, w_ref, b_ref, wg_ref, bg_ref, o_ref, sacc):
        i = pl.program_id(0)
        nd = lax.rsqrt(jnp.maximum(din_ref[...], 1.0))
        h = jnp.dot(a_ref[0] * nd, w_ref[0:CW, :],
                    preferred_element_type=jnp.float32)
        for c in range(1, nci):
            h = h + jnp.dot(a_ref[c] * nd, w_ref[c * CW:(c + 1) * CW, :],
                            preferred_element_type=jnp.float32)
        h = jnp.maximum(h + b_ref[...], 0.0)
        part = jnp.sum(h, axis=0, keepdims=True)

        @pl.when(i == 0)
        def _():
            sacc[...] = part

        @pl.when(i > 0)
        def _():
            sacc[...] = sacc[...] + part

        @pl.when(i == nblk - 1)
        def _():
            o_ref[...] = jnp.dot(sacc[...] * (1.0 / N), wg_ref[...],
                                 preferred_element_type=jnp.float32) + bg_ref[...]

    return pl.pallas_call(
        body,
        grid=(nblk,),
        in_specs=[
            pl.BlockSpec((nci, RB, CW), lambda i: (0, i, 0)),
            pl.BlockSpec((RB, 1), lambda i: (i, 0)),
            pl.BlockSpec((H, H), lambda i: (0, 0)),
            pl.BlockSpec((1, H), lambda i: (0, 0)),
            pl.BlockSpec((H, D_OUT), lambda i: (0, 0)),
            pl.BlockSpec((1, D_OUT), lambda i: (0, 0)),
        ],
        out_specs=pl.BlockSpec((1, D_OUT), lambda i: (0, 0)),
        out_shape=jax.ShapeDtypeStruct((1, D_OUT), jnp.float32),
        scratch_shapes=[pltpu.VMEM((1, H), jnp.float32)],
    )(agg, din_col, W1, b1r, Wg, bgr)


def kernel(x, edge_index, W0, b0, W1, b1, Wg, bg):
    assert x.shape == (N, D_IN) and edge_index.shape == (2, E)

    ei2 = edge_index.reshape(2, 16, NB, B)

    deg = _sc_degrees(ei2)
    dout_col = deg[:N].reshape(N, 1)
    din_col = deg[NPAD:NPAD + N].reshape(N, 1)

    nc1 = D_IN // CW
    nc2 = H // CW
    xs = _tc_scale(x, dout_col)
    agg1 = _sc_aggregate(ei2, xs.reshape(nc1 * N, CW), nc1)
    h1s = _tc_layer1(agg1.reshape(nc1, N, CW), din_col, dout_col,
                     W0, b0.reshape(1, H))
    agg2 = _sc_aggregate(ei2, h1s.reshape(nc2 * N, CW), nc2)
    out = _tc_readout(agg2.reshape(nc2, N, CW), din_col,
                      W1, b1.reshape(1, H), Wg, bg.reshape(1, D_OUT))
    return out


# speedup vs baseline: 8.6077x; 1.5814x over previous
import functools

import jax
import jax.numpy as jnp
from jax import lax
from jax.experimental import pallas as pl
from jax.experimental.pallas import tpu as pltpu
from jax.experimental.pallas import tpu_sc as plsc

N = 10000
E = 160000
D_IN = 256
H = 512
D_OUT = 256

NPAD = 10240
RPT = 640
B = 80
NB = 125
RB = 2000
CW = 128

_MESH = dict(core_axis_name="c", subcore_axis_name="s")


def _sc_degrees(ei2):
    @functools.partial(
        pl.kernel,
        out_type=jax.ShapeDtypeStruct((2 * NPAD,), jnp.float32),
        mesh=plsc.VectorSubcoreMesh(**_MESH),
        scratch_types=[
            pltpu.VMEM((NB, B), jnp.int32),
            pltpu.VMEM((B,), jnp.float32),
            pltpu.VMEM((RPT,), jnp.float32),
            pltpu.VMEM_SHARED((NPAD,), jnp.float32),
            pltpu.SemaphoreType.DMA,
        ],
    )
    def deg_kernel(ei_hbm, deg_hbm, idx, ones, fbuf, acc, sem):
        cid = lax.axis_index("c")
        sid = lax.axis_index("s")

        pltpu.sync_copy(ei_hbm.at[cid, sid], idx)

        def fill_ones(j, _):
            ones[pl.ds(j * 16, 16)] = jnp.full((16,), 1.0, jnp.float32)
            return 0
        lax.fori_loop(0, B // 16, fill_ones, 0)

        def fill_zero(j, _):
            fbuf[pl.ds(j * 16, 16)] = jnp.zeros((16,), jnp.float32)
            return 0
        lax.fori_loop(0, RPT // 16, fill_zero, 0)

        r0 = sid * RPT
        pltpu.sync_copy(fbuf, acc.at[pl.ds(r0, RPT)])
        plsc.subcore_barrier()

        def fire(j, _):
            pltpu.async_copy(ones, acc.at[idx.at[j]], sem, add=True)
            return 0
        lax.fori_loop(0, NB, fire, 0)

        def drain(j, _):
            pltpu.make_async_copy(ones, acc.at[idx.at[j]], sem).wait()
            return 0
        lax.fori_loop(0, NB, drain, 0)
        plsc.subcore_barrier()

        pltpu.sync_copy(acc.at[pl.ds(r0, RPT)], fbuf)
        pltpu.sync_copy(fbuf, deg_hbm.at[pl.ds(cid * NPAD + r0, RPT)])

    return deg_kernel(ei2)


def _sc_aggregate(ei5, tab, nc):
    ncps = nc // 2
    GN = NB // 5

    @functools.partial(
        pl.kernel,
        out_type=jax.ShapeDtypeStruct((nc * N, CW), jnp.float32),
        mesh=plsc.VectorSubcoreMesh(**_MESH),
        scratch_types=[
            pltpu.VMEM((GN, B), jnp.int32),
            pltpu.VMEM((GN, B), jnp.int32),
            pltpu.VMEM((GN, B), jnp.int32),
            pltpu.VMEM((GN, B), jnp.int32),
            pltpu.VMEM((B, CW), jnp.float32),
            pltpu.VMEM((B, CW), jnp.float32),
            pltpu.VMEM_SHARED((NPAD, CW), jnp.float32),
            pltpu.SemaphoreType.DMA,
            pltpu.SemaphoreType.DMA,
            pltpu.SemaphoreType.DMA,
        ],
    )
    def agg_kernel(ei_hbm, tab_hbm, out_hbm, gis0, gis1, gid0, gid1,
                   rows, rows_b, acc, sem, sem_b, isem):
        cid = lax.axis_index("c")
        sid = lax.axis_index("s")

        def add_off(gis, off):
            def body(r, _):
                for k in range(B // 16):
                    sl = (r, pl.ds(k * 16, 16))
                    gis[sl] = gis[sl] + off
                return 0
            lax.fori_loop(0, GN, body, 0)

        def fill_zero(j, _):
            for k in range(CW // 16):
                rows[j, pl.ds(k * 16, 16)] = jnp.zeros((16,), jnp.float32)
            return 0

        r0 = sid * RPT
        for ci in range(ncps):
            off = (cid * ncps + ci) * N
            lax.fori_loop(0, B, fill_zero, 0)
            for blk in range(RPT // B):
                pltpu.sync_copy(rows, acc.at[pl.ds(r0 + blk * B, B)])
            plsc.subcore_barrier()

            pltpu.sync_copy(ei_hbm.at[0, sid, 0], gis0)
            pltpu.sync_copy(ei_hbm.at[1, sid, 0], gid0)
            add_off(gis0, off)

            for g in range(5):
                gis, gid = (gis0, gid0) if g % 2 == 0 else (gis1, gid1)
                nis, nid = (gis0, gid0) if g % 2 == 1 else (gis1, gid1)
                if g + 1 < 5:
                    pltpu.async_copy(ei_hbm.at[0, sid, g + 1], nis, isem)
                    pltpu.async_copy(ei_hbm.at[1, sid, g + 1], nid, isem)

                pltpu.async_copy(tab_hbm.at[gis.at[0]], rows, sem)

                def dbl(p, _):
                    j0 = 2 * p
                    pltpu.async_copy(tab_hbm.at[gis.at[j0 + 1]], rows_b, sem_b)
                    pltpu.make_async_copy(tab_hbm.at[gis.at[j0]], rows, sem).wait()
                    pltpu.sync_copy(rows, acc.at[gid.at[j0]], add=True)
                    pltpu.async_copy(tab_hbm.at[gis.at[j0 + 2]], rows, sem)
                    pltpu.make_async_copy(tab_hbm.at[gis.at[j0 + 1]], rows_b, sem_b).wait()
                    pltpu.sync_copy(rows_b, acc.at[gid.at[j0 + 1]], add=True)
                    return 0
                lax.fori_loop(0, (GN - 1) // 2, dbl, 0)

                pltpu.make_async_copy(tab_hbm.at[gis.at[GN - 1]], rows, sem).wait()
                pltpu.sync_copy(rows, acc.at[gid.at[GN - 1]], add=True)

                if g + 1 < 5:
                    pltpu.make_async_copy(ei_hbm.at[0, sid, g + 1], nis, isem).wait()
                    pltpu.make_async_copy(ei_hbm.at[1, sid, g + 1], nid, isem).wait()
                    add_off(nis, off)
            plsc.subcore_barrier()

            chunk_base = (cid * ncps + ci) * N
            for blk in range(RPT // B):
                rr = r0 + blk * B

                @pl.when(rr < N)
                def _():
                    pltpu.sync_copy(acc.at[pl.ds(rr, B)], rows)
                    pltpu.sync_copy(rows, out_hbm.at[pl.ds(chunk_base + rr, B)])

    return agg_kernel(ei5, tab)


def _tc_scale(x, dout_col):
    nc = D_IN // CW

    def body(x_ref, d_ref, o_ref):
        nrm = lax.rsqrt(jnp.maximum(d_ref[...], 1.0))
        xs = x_ref[...] * nrm
        for c in range(nc):
            o_ref[c] = xs[:, c * CW:(c + 1) * CW]

    return pl.pallas_call(
        body,
        grid=(N // RB,),
        in_specs=[
            pl.BlockSpec((RB, D_IN), lambda i: (i, 0)),
            pl.BlockSpec((RB, 1), lambda i: (i, 0)),
        ],
        out_specs=pl.BlockSpec((nc, RB, CW), lambda i: (0, i, 0)),
        out_shape=jax.ShapeDtypeStruct((nc, N, CW), jnp.float32),
    )(x, dout_col)


def _tc_layer1(agg, din_col, dout_col, W0, b0r):
    nci = D_IN // CW
    nco = H // CW

    def body(a_ref, din_ref, dout_ref, w_ref, b_ref, o_ref):
        nd = lax.rsqrt(jnp.maximum(din_ref[...], 1.0))
        ns = lax.rsqrt(jnp.maximum(dout_ref[...], 1.0))
        h = jnp.dot(a_ref[0] * nd, w_ref[0:CW, :],
                    preferred_element_type=jnp.float32)
        for c in range(1, nci):
            h = h + jnp.dot(a_ref[c] * nd, w_ref[c * CW:(c + 1) * CW, :],
                            preferred_element_type=jnp.float32)
        h = jnp.maximum(h + b_ref[...], 0.0) * ns
        for c in range(nco):
            o_ref[c] = h[:, c * CW:(c + 1) * CW]

    return pl.pallas_call(
        body,
        grid=(N // RB,),
        in_specs=[
            pl.BlockSpec((nci, RB, CW), lambda i: (0, i, 0)),
            pl.BlockSpec((RB, 1), lambda i: (i, 0)),
            pl.BlockSpec((RB, 1), lambda i: (i, 0)),
            pl.BlockSpec((D_IN, H), lambda i: (0, 0)),
            pl.BlockSpec((1, H), lambda i: (0, 0)),
        ],
        out_specs=pl.BlockSpec((nco, RB, CW), lambda i: (0, i, 0)),
        out_shape=jax.ShapeDtypeStruct((nco, N, CW), jnp.float32),
    )(agg, din_col, dout_col, W0, b0r)


def _tc_readout(agg, din_col, W1, b1r, Wg, bgr):
    nblk = N // RB

    nci = H // CW

    def body(a_ref, din_ref, w_ref, b_ref, wg_ref, bg_ref, o_ref, sacc):
        i = pl.program_id(0)
        nd = lax.rsqrt(jnp.maximum(din_ref[...], 1.0))
        h = jnp.dot(a_ref[0] * nd, w_ref[0:CW, :],
                    preferred_element_type=jnp.float32)
        for c in range(1, nci):
            h = h + jnp.dot(a_ref[c] * nd, w_ref[c * CW:(c + 1) * CW, :],
                            preferred_element_type=jnp.float32)
        h = jnp.maximum(h + b_ref[...], 0.0)
        part = jnp.sum(h, axis=0, keepdims=True)

        @pl.when(i == 0)
        def _():
            sacc[...] = part

        @pl.when(i > 0)
        def _():
            sacc[...] = sacc[...] + part

        @pl.when(i == nblk - 1)
        def _():
            o_ref[...] = jnp.dot(sacc[...] * (1.0 / N), wg_ref[...],
                                 preferred_element_type=jnp.float32) + bg_ref[...]

    return pl.pallas_call(
        body,
        grid=(nblk,),
        in_specs=[
            pl.BlockSpec((nci, RB, CW), lambda i: (0, i, 0)),
            pl.BlockSpec((RB, 1), lambda i: (i, 0)),
            pl.BlockSpec((H, H), lambda i: (0, 0)),
            pl.BlockSpec((1, H), lambda i: (0, 0)),
            pl.BlockSpec((H, D_OUT), lambda i: (0, 0)),
            pl.BlockSpec((1, D_OUT), lambda i: (0, 0)),
        ],
        out_specs=pl.BlockSpec((1, D_OUT), lambda i: (0, 0)),
        out_shape=jax.ShapeDtypeStruct((1, D_OUT), jnp.float32),
        scratch_shapes=[pltpu.VMEM((1, H), jnp.float32)],
    )(agg, din_col, W1, b1r, Wg, bgr)


def kernel(x, edge_index, W0, b0, W1, b1, Wg, bg):
    assert x.shape == (N, D_IN) and edge_index.shape == (2, E)

    ei2 = edge_index.reshape(2, 16, NB, B)
    ei5 = edge_index.reshape(2, 16, 5, NB // 5, B)

    deg = _sc_degrees(ei2)
    dout_col = deg[:N].reshape(N, 1)
    din_col = deg[NPAD:NPAD + N].reshape(N, 1)

    nc1 = D_IN // CW
    nc2 = H // CW
    xs = _tc_scale(x, dout_col)
    agg1 = _sc_aggregate(ei5, xs.reshape(nc1 * N, CW), nc1)
    h1s = _tc_layer1(agg1.reshape(nc1, N, CW), din_col, dout_col,
                     W0, b0.reshape(1, H))
    agg2 = _sc_aggregate(ei5, h1s.reshape(nc2 * N, CW), nc2)
    out = _tc_readout(agg2.reshape(nc2, N, CW), din_col,
                      W1, b1.reshape(1, H), Wg, bg.reshape(1, D_OUT))
    return out


# speedup vs baseline: 8.8736x; 1.0309x over previous
import functools

import jax
import jax.numpy as jnp
from jax import lax
from jax.experimental import pallas as pl
from jax.experimental.pallas import tpu as pltpu
from jax.experimental.pallas import tpu_sc as plsc

N = 10000
E = 160000
D_IN = 256
H = 512
D_OUT = 256

NPAD = 10240
RPT = 640
B = 80
NB = 125
RB = 2000
CW = 128

_MESH = dict(core_axis_name="c", subcore_axis_name="s")


def _sc_degrees(ei2):
    @functools.partial(
        pl.kernel,
        out_type=jax.ShapeDtypeStruct((2 * NPAD,), jnp.float32),
        mesh=plsc.VectorSubcoreMesh(**_MESH),
        scratch_types=[
            pltpu.VMEM((NB, B), jnp.int32),
            pltpu.VMEM((B,), jnp.float32),
            pltpu.VMEM((RPT,), jnp.float32),
            pltpu.VMEM_SHARED((NPAD,), jnp.float32),
            pltpu.SemaphoreType.DMA,
        ],
    )
    def deg_kernel(ei_hbm, deg_hbm, idx, ones, fbuf, acc, sem):
        cid = lax.axis_index("c")
        sid = lax.axis_index("s")

        pltpu.sync_copy(ei_hbm.at[cid, sid], idx)

        def fill_ones(j, _):
            ones[pl.ds(j * 16, 16)] = jnp.full((16,), 1.0, jnp.float32)
            return 0
        lax.fori_loop(0, B // 16, fill_ones, 0)

        def fill_zero(j, _):
            fbuf[pl.ds(j * 16, 16)] = jnp.zeros((16,), jnp.float32)
            return 0
        lax.fori_loop(0, RPT // 16, fill_zero, 0)

        r0 = sid * RPT
        pltpu.sync_copy(fbuf, acc.at[pl.ds(r0, RPT)])
        plsc.subcore_barrier()

        def fire(j, _):
            pltpu.async_copy(ones, acc.at[idx.at[j]], sem, add=True)
            return 0
        lax.fori_loop(0, NB, fire, 0)

        def drain(j, _):
            pltpu.make_async_copy(ones, acc.at[idx.at[j]], sem).wait()
            return 0
        lax.fori_loop(0, NB, drain, 0)
        plsc.subcore_barrier()

        pltpu.sync_copy(acc.at[pl.ds(r0, RPT)], fbuf)
        pltpu.sync_copy(fbuf, deg_hbm.at[pl.ds(cid * NPAD + r0, RPT)])

    return deg_kernel(ei2)


def _sc_aggregate(ei5, tab, nc):
    ncps = nc // 2
    GN = NB // 5

    @functools.partial(
        pl.kernel,
        out_type=jax.ShapeDtypeStruct((nc * N, CW), jnp.float32),
        mesh=plsc.VectorSubcoreMesh(**_MESH),
        scratch_types=[
            pltpu.VMEM((GN, B), jnp.int32),
            pltpu.VMEM((GN, B), jnp.int32),
            pltpu.VMEM((GN, B), jnp.int32),
            pltpu.VMEM((GN, B), jnp.int32),
            pltpu.VMEM((B, CW), jnp.float32),
            pltpu.VMEM((B, CW), jnp.float32),
            pltpu.VMEM_SHARED((NPAD, CW), jnp.float32),
            pltpu.SemaphoreType.DMA,
            pltpu.SemaphoreType.DMA,
            pltpu.SemaphoreType.DMA,
        ],
    )
    def agg_kernel(ei_hbm, tab_hbm, out_hbm, gis0, gis1, gid0, gid1,
                   rows, rows_b, acc, sem, sem_b, isem):
        cid = lax.axis_index("c")
        sid = lax.axis_index("s")

        def add_off(gis, off):
            def body(r, _):
                for k in range(B // 16):
                    sl = (r, pl.ds(k * 16, 16))
                    gis[sl] = gis[sl] + off
                return 0
            lax.fori_loop(0, GN, body, 0)

        def fill_zero(j, _):
            for k in range(CW // 16):
                rows[j, pl.ds(k * 16, 16)] = jnp.zeros((16,), jnp.float32)
            return 0

        r0 = sid * RPT
        nblk = RPT // B
        for ci in range(ncps):
            off = (cid * ncps + ci) * N
            pltpu.async_copy(ei_hbm.at[0, sid, 0], gis0, isem)
            pltpu.async_copy(ei_hbm.at[1, sid, 0], gid0, isem)
            lax.fori_loop(0, B, fill_zero, 0)
            for blk in range(nblk):
                pltpu.async_copy(rows, acc.at[pl.ds(r0 + blk * B, B)], sem)
            for blk in range(nblk):
                pltpu.make_async_copy(rows, acc.at[pl.ds(r0 + blk * B, B)], sem).wait()
            pltpu.make_async_copy(ei_hbm.at[0, sid, 0], gis0, isem).wait()
            pltpu.make_async_copy(ei_hbm.at[1, sid, 0], gid0, isem).wait()
            plsc.subcore_barrier()

            add_off(gis0, off)

            for g in range(5):
                gis, gid = (gis0, gid0) if g % 2 == 0 else (gis1, gid1)
                nis, nid = (gis0, gid0) if g % 2 == 1 else (gis1, gid1)
                if g + 1 < 5:
                    pltpu.async_copy(ei_hbm.at[0, sid, g + 1], nis, isem)
                    pltpu.async_copy(ei_hbm.at[1, sid, g + 1], nid, isem)

                pltpu.async_copy(tab_hbm.at[gis.at[0]], rows, sem)

                def dbl(p, _):
                    j0 = 2 * p
                    pltpu.async_copy(tab_hbm.at[gis.at[j0 + 1]], rows_b, sem_b)
                    pltpu.make_async_copy(tab_hbm.at[gis.at[j0]], rows, sem).wait()
                    pltpu.sync_copy(rows, acc.at[gid.at[j0]], add=True)
                    pltpu.async_copy(tab_hbm.at[gis.at[j0 + 2]], rows, sem)
                    pltpu.make_async_copy(tab_hbm.at[gis.at[j0 + 1]], rows_b, sem_b).wait()
                    pltpu.sync_copy(rows_b, acc.at[gid.at[j0 + 1]], add=True)
                    return 0
                lax.fori_loop(0, (GN - 1) // 2, dbl, 0)

                pltpu.make_async_copy(tab_hbm.at[gis.at[GN - 1]], rows, sem).wait()
                pltpu.sync_copy(rows, acc.at[gid.at[GN - 1]], add=True)

                if g + 1 < 5:
                    pltpu.make_async_copy(ei_hbm.at[0, sid, g + 1], nis, isem).wait()
                    pltpu.make_async_copy(ei_hbm.at[1, sid, g + 1], nid, isem).wait()
                    add_off(nis, off)
            plsc.subcore_barrier()

            chunk_base = (cid * ncps + ci) * N
            limit = jnp.minimum(jnp.int32(N), r0 + RPT)
            for blk in range(nblk):
                rr = r0 + blk * B
                buf = rows if blk % 2 == 0 else rows_b
                fsem = sem if blk % 2 == 0 else sem_b

                @pl.when(rr < N)
                def _():
                    if blk >= 2:
                        pltpu.make_async_copy(
                            buf, out_hbm.at[pl.ds(chunk_base + rr - 2 * B, B)],
                            fsem).wait()
                    pltpu.sync_copy(acc.at[pl.ds(rr, B)], buf)
                    pltpu.async_copy(buf, out_hbm.at[pl.ds(chunk_base + rr, B)], fsem)
            for blk in range(nblk):
                rr = r0 + blk * B
                buf = rows if blk % 2 == 0 else rows_b
                fsem = sem if blk % 2 == 0 else sem_b

                @pl.when((rr < N) & (rr + 2 * B >= limit))
                def _():
                    pltpu.make_async_copy(
                        buf, out_hbm.at[pl.ds(chunk_base + rr, B)], fsem).wait()

    return agg_kernel(ei5, tab)


def _tc_scale(x, dout_col):
    nc = D_IN // CW

    def body(x_ref, d_ref, o_ref):
        nrm = lax.rsqrt(jnp.maximum(d_ref[...], 1.0))
        xs = x_ref[...] * nrm
        for c in range(nc):
            o_ref[c] = xs[:, c * CW:(c + 1) * CW]

    return pl.pallas_call(
        body,
        grid=(N // RB,),
        in_specs=[
            pl.BlockSpec((RB, D_IN), lambda i: (i, 0)),
            pl.BlockSpec((RB, 1), lambda i: (i, 0)),
        ],
        out_specs=pl.BlockSpec((nc, RB, CW), lambda i: (0, i, 0)),
        out_shape=jax.ShapeDtypeStruct((nc, N, CW), jnp.float32),
    )(x, dout_col)


def _tc_layer1(agg, din_col, dout_col, W0, b0r):
    nci = D_IN // CW
    nco = H // CW

    def body(a_ref, din_ref, dout_ref, w_ref, b_ref, o_ref):
        nd = lax.rsqrt(jnp.maximum(din_ref[...], 1.0))
        ns = lax.rsqrt(jnp.maximum(dout_ref[...], 1.0))
        h = jnp.dot(a_ref[0] * nd, w_ref[0:CW, :],
                    preferred_element_type=jnp.float32)
        for c in range(1, nci):
            h = h + jnp.dot(a_ref[c] * nd, w_ref[c * CW:(c + 1) * CW, :],
                            preferred_element_type=jnp.float32)
        h = jnp.maximum(h + b_ref[...], 0.0) * ns
        for c in range(nco):
            o_ref[c] = h[:, c * CW:(c + 1) * CW]

    return pl.pallas_call(
        body,
        grid=(N // RB,),
        in_specs=[
            pl.BlockSpec((nci, RB, CW), lambda i: (0, i, 0)),
            pl.BlockSpec((RB, 1), lambda i: (i, 0)),
            pl.BlockSpec((RB, 1), lambda i: (i, 0)),
            pl.BlockSpec((D_IN, H), lambda i: (0, 0)),
            pl.BlockSpec((1, H), lambda i: (0, 0)),
        ],
        out_specs=pl.BlockSpec((nco, RB, CW), lambda i: (0, i, 0)),
        out_shape=jax.ShapeDtypeStruct((nco, N, CW), jnp.float32),
    )(agg, din_col, dout_col, W0, b0r)


def _tc_readout(agg, din_col, W1, b1r, Wg, bgr):
    nblk = N // RB

    nci = H // CW

    def body(a_ref, din_ref, w_ref, b_ref, wg_ref, bg_ref, o_ref, sacc):
        i = pl.program_id(0)
        nd = lax.rsqrt(jnp.maximum(din_ref[...], 1.0))
        h = jnp.dot(a_ref[0] * nd, w_ref[0:CW, :],
                    preferred_element_type=jnp.float32)
        for c in range(1, nci):
            h = h + jnp.dot(a_ref[c] * nd, w_ref[c * CW:(c + 1) * CW, :],
                            preferred_element_type=jnp.float32)
        h = jnp.maximum(h + b_ref[...], 0.0)
        part = jnp.sum(h, axis=0, keepdims=True)

        @pl.when(i == 0)
        def _():
            sacc[...] = part

        @pl.when(i > 0)
        def _():
            sacc[...] = sacc[...] + part

        @pl.when(i == nblk - 1)
        def _():
            o_ref[...] = jnp.dot(sacc[...] * (1.0 / N), wg_ref[...],
                                 preferred_element_type=jnp.float32) + bg_ref[...]

    return pl.pallas_call(
        body,
        grid=(nblk,),
        in_specs=[
            pl.BlockSpec((nci, RB, CW), lambda i: (0, i, 0)),
            pl.BlockSpec((RB, 1), lambda i: (i, 0)),
            pl.BlockSpec((H, H), lambda i: (0, 0)),
            pl.BlockSpec((1, H), lambda i: (0, 0)),
            pl.BlockSpec((H, D_OUT), lambda i: (0, 0)),
            pl.BlockSpec((1, D_OUT), lambda i: (0, 0)),
        ],
        out_specs=pl.BlockSpec((1, D_OUT), lambda i: (0, 0)),
        out_shape=jax.ShapeDtypeStruct((1, D_OUT), jnp.float32),
        scratch_shapes=[pltpu.VMEM((1, H), jnp.float32)],
    )(agg, din_col, W1, b1r, Wg, bgr)


def kernel(x, edge_index, W0, b0, W1, b1, Wg, bg):
    assert x.shape == (N, D_IN) and edge_index.shape == (2, E)

    ei2 = edge_index.reshape(2, 16, NB, B)
    ei5 = edge_index.reshape(2, 16, 5, NB // 5, B)

    deg = _sc_degrees(ei2)
    dout_col = deg[:N].reshape(N, 1)
    din_col = deg[NPAD:NPAD + N].reshape(N, 1)

    nc1 = D_IN // CW
    nc2 = H // CW
    xs = _tc_scale(x, dout_col)
    agg1 = _sc_aggregate(ei5, xs.reshape(nc1 * N, CW), nc1)
    h1s = _tc_layer1(agg1.reshape(nc1, N, CW), din_col, dout_col,
                     W0, b0.reshape(1, H))
    agg2 = _sc_aggregate(ei5, h1s.reshape(nc2 * N, CW), nc2)
    out = _tc_readout(agg2.reshape(nc2, N, CW), din_col,
                      W1, b1.reshape(1, H), Wg, bg.reshape(1, D_OUT))
    return out
